# Initial kernel scaffold; baseline (speedup 1.0000x reference)
import jax
import jax.numpy as jnp
from jax.experimental import pallas as pl


def kernel(x, edge_index, W):
    raise NotImplementedError("write your pallas kernel here")


# speedup vs baseline: 8.3772x; 8.3772x over previous
import jax
import jax.numpy as jnp
from jax import lax
from jax.experimental import pallas as pl
from jax.experimental.pallas import tpu as pltpu
from jax.experimental.pallas import tpu_sc as plsc

N = 10000
E = 320000
D = 128
NC, NS, L = 2, 16, 16
NW = NC * NS
NP = 12288
RPW = NP // NW
RPT = NP // NS

CHA = 79
CHC = 82

_MESH = plsc.VectorSubcoreMesh(
    core_axis_name="c", subcore_axis_name="s", num_cores=NC, num_subcores=NS)


def _wid():
    return lax.axis_index("s") * NC + lax.axis_index("c")


def _hist_body(dst_hbm, val_hbm, zf_hbm, hist_hbm, dst_v, val_v, bounce_v,
               hist_sh):
    c = lax.axis_index("c")
    s = lax.axis_index("s")
    w = _wid()

    @pl.when(s == 0)
    def _():
        pltpu.sync_copy(zf_hbm, hist_sh)

    pltpu.sync_copy(dst_hbm.at[w], dst_v)
    pltpu.sync_copy(val_hbm.at[w], val_v)
    plsc.subcore_barrier()

    def body(j, carry):
        pltpu.sync_copy(val_v.at[j], hist_sh.at[dst_v.at[j]], add=True)
        return carry

    lax.fori_loop(0, CHA, body, 0)
    plsc.subcore_barrier()

    @pl.when(s == 0)
    def _():
        pltpu.sync_copy(hist_sh, bounce_v)
        pltpu.sync_copy(bounce_v, hist_hbm.at[c])


def _make_hist():
    return pl.kernel(
        _hist_body,
        out_type=jax.ShapeDtypeStruct((NC, NP), jnp.float32),
        mesh=_MESH,
        scratch_types=[
            pltpu.VMEM((CHA, 128), jnp.int32),
            pltpu.VMEM((CHA, 128), jnp.float32),
            pltpu.VMEM((NP,), jnp.float32),
            pltpu.VMEM_SHARED((NP,), jnp.float32),
        ],
    )


def _feat_body(hist_hbm, x_hbm, feat_hbm, normb_hbm, h0_v, h1_v, norm_v, x_v,
               nb_v):
    w = _wid()
    base = w * RPW
    pltpu.sync_copy(hist_hbm.at[0].at[pl.ds(base, RPW)], h0_v)
    pltpu.sync_copy(hist_hbm.at[1].at[pl.ds(base, RPW)], h1_v)
    pltpu.sync_copy(x_hbm.at[pl.ds(base * D, RPW * D)], x_v)

    def nbody(v, carry):
        deg = h0_v[pl.ds(v * L, L)] + h1_v[pl.ds(v * L, L)] + 1.0
        i = lax.bitcast_convert_type(deg, jnp.int32)
        i = 0x5F3759DF - lax.shift_right_logical(i, 1)
        y = lax.bitcast_convert_type(i, jnp.float32)
        hh = deg * 0.5
        y = y * (1.5 - hh * y * y)
        y = y * (1.5 - hh * y * y)
        y = y * (1.5 - hh * y * y)
        norm_v[pl.ds(v * L, L)] = y
        return carry

    lax.fori_loop(0, RPW // L, nbody, 0)

    def gbody(g, carry):
        nv = norm_v[pl.ds(g * L, L)]
        for lane in range(L):
            idx = (jnp.zeros((L,), jnp.int32) + lane)[:, None]
            nb = lax.gather(
                nv, idx,
                lax.GatherDimensionNumbers(offset_dims=(),
                                           collapsed_slice_dims=(0,),
                                           start_index_map=(0,)),
                slice_sizes=(1,),
                mode=lax.GatherScatterMode.PROMISE_IN_BOUNDS)
            for k in range(D // L):
                off = (g * L + lane) * D + k * L
                x_v[pl.ds(off, L)] = x_v[pl.ds(off, L)] * nb
                nb_v[pl.ds(off, L)] = nb
        return carry

    lax.fori_loop(0, RPW // L, gbody, 0)

    pltpu.sync_copy(x_v, feat_hbm.at[pl.ds(base * D, RPW * D)])
    pltpu.sync_copy(nb_v, normb_hbm.at[pl.ds(base * D, RPW * D)])


def _make_feat():
    return pl.kernel(
        _feat_body,
        out_type=(jax.ShapeDtypeStruct((NP * D,), jnp.float32),
                  jax.ShapeDtypeStruct((NP * D,), jnp.float32)),
        mesh=_MESH,
        scratch_types=[
            pltpu.VMEM((RPW,), jnp.float32),
            pltpu.VMEM((RPW,), jnp.float32),
            pltpu.VMEM((RPW,), jnp.float32),
            pltpu.VMEM((RPW * D,), jnp.float32),
            pltpu.VMEM((RPW * D,), jnp.float32),
        ],
    )


DH = D // 2


def _agg_body(src_hbm, dst_hbm, feat0_hbm, feat1_hbm, z_hbm, pout_hbm,
              src_v, dst_v, buf_a, buf_b, sem_a, sem_b, agg_sh):
    c = lax.axis_index("c")
    s = lax.axis_index("s")
    w = _wid()

    pltpu.sync_copy(src_hbm.at[w], src_v)
    pltpu.sync_copy(dst_hbm.at[w], dst_v)

    for k, feat_hbm in ((0, feat0_hbm), (1, feat1_hbm)):
        @pl.when(s == 0)
        def _():
            pltpu.sync_copy(z_hbm, agg_sh)

        plsc.subcore_barrier()

        pltpu.async_copy(feat_hbm.at[src_v.at[0]], buf_a, sem_a)
        pltpu.async_copy(feat_hbm.at[src_v.at[1]], buf_b, sem_b)

        def body(t, carry):
            for b, (buf, sem) in enumerate(((buf_a, sem_a), (buf_b, sem_b))):
                j = 2 * t + b
                pltpu.make_async_copy(feat_hbm.at[src_v.at[j]], buf,
                                      sem).wait()
                pltpu.sync_copy(buf, agg_sh.at[dst_v.at[j]], add=True)

                @pl.when(j + 2 < CHC)
                def _():
                    pltpu.async_copy(feat_hbm.at[src_v.at[j + 2]], buf, sem)
            return carry

        lax.fori_loop(0, CHC // 2, body, 0)
        plsc.subcore_barrier()

        for q in range(RPT // 128):
            row0 = s * RPT + q * 128
            pltpu.sync_copy(agg_sh.at[pl.ds(row0, 128)], buf_a)
            pltpu.sync_copy(buf_a, pout_hbm.at[c].at[k].at[pl.ds(row0, 128)])
        plsc.subcore_barrier()


def _make_agg():
    return pl.kernel(
        _agg_body,
        out_type=jax.ShapeDtypeStruct((NC, 2, NP, DH), jnp.float32),
        mesh=_MESH,
        scratch_types=[
            pltpu.VMEM((CHC, 128), jnp.int32),
            pltpu.VMEM((CHC, 128), jnp.int32),
            pltpu.VMEM((128, DH), jnp.float32),
            pltpu.VMEM((128, DH), jnp.float32),
            pltpu.SemaphoreType.DMA,
            pltpu.SemaphoreType.DMA,
            pltpu.VMEM_SHARED((NP, DH), jnp.float32),
        ],
        compiler_params=pltpu.CompilerParams(use_tc_tiling_on_sc=False),
    )


_BD = 512


def _mm_body(p0_ref, p1_ref, nb_ref, w_ref, o_ref):
    a = (p0_ref[...] + p1_ref[...]) * nb_ref[...]
    o_ref[...] = lax.dot_general(a, w_ref[...], (((1,), (1,)), ((), ())),
                                 preferred_element_type=jnp.float32)


def _matmul(p0, p1, normb, w):
    return pl.pallas_call(
        _mm_body,
        grid=(NP // _BD,),
        in_specs=[
            pl.BlockSpec((_BD, D), lambda i: (i, 0)),
            pl.BlockSpec((_BD, D), lambda i: (i, 0)),
            pl.BlockSpec((_BD, D), lambda i: (i, 0)),
            pl.BlockSpec((D, D), lambda i: (0, 0)),
        ],
        out_specs=pl.BlockSpec((_BD, D), lambda i: (i, 0)),
        out_shape=jax.ShapeDtypeStruct((NP, D), jnp.float32),
    )(p0, p1, normb, w)


NR = NP // 128
NP2 = NP + 128


def _pos_body(h_ref, u_ref, s_ref, o_ref):
    p = ((h_ref[0] + h_ref[1]) > 0.0).astype(jnp.float32)
    incl = lax.dot_general(p, u_ref[...], (((1,), (0,)), ((), ())),
                           precision=lax.Precision.HIGHEST,
                           preferred_element_type=jnp.float32)
    rows = incl[:, 127:128]
    rowpref = lax.dot_general(s_ref[...], rows, (((1,), (0,)), ((), ())),
                              precision=lax.Precision.HIGHEST,
                              preferred_element_type=jnp.float32)
    excl = incl - p + rowpref
    col = lax.broadcasted_iota(jnp.int32, (NR, 128), 1)
    o_ref[...] = jnp.where(p > 0.0, excl.astype(jnp.int32), NP + col)


def _positions(h3, u128, s96):
    return pl.pallas_call(
        _pos_body,
        grid=(1,),
        in_specs=[
            pl.BlockSpec((2, NR, 128), lambda i: (0, 0, 0)),
            pl.BlockSpec((128, 128), lambda i: (0, 0)),
            pl.BlockSpec((NR, NR), lambda i: (0, 0)),
        ],
        out_specs=pl.BlockSpec((NR, 128), lambda i: (0, 0)),
        out_shape=jax.ShapeDtypeStruct((NR, 128), jnp.int32),
    )(h3, u128, s96)


RPS = 8


def _gather_body(midx_hbm, vals_hbm, src2d_hbm, zi_hbm, fin_hbm, pidx_v,
                 pval_v, idx_v, rows_v, sem, dn_sh):
    s = lax.axis_index("s")
    w = _wid()

    @pl.when(s == 0)
    def _():
        pltpu.sync_copy(zi_hbm, dn_sh)

    @pl.when(s < NR // RPS)
    def _():
        pltpu.sync_copy(midx_hbm.at[pl.ds(s * RPS, RPS)], pidx_v)
        pltpu.sync_copy(vals_hbm.at[pl.ds(s * RPS, RPS)], pval_v)

    plsc.subcore_barrier()

    @pl.when(s < NR // RPS)
    def _():
        for r in range(RPS):
            pltpu.sync_copy(pval_v.at[r], dn_sh.at[pidx_v.at[r]])

    plsc.subcore_barrier()

    base = w * RPW
    pltpu.sync_copy(dn_sh.at[pl.ds(base, RPW)], idx_v)
    for off in (0, 128, 256):
        pltpu.async_copy(src2d_hbm.at[idx_v.at[pl.ds(off, 128)]],
                         rows_v, sem).wait()
        pltpu.sync_copy(rows_v, fin_hbm.at[pl.ds(base + off, 128)])


def _make_gather():
    return pl.kernel(
        _gather_body,
        out_type=jax.ShapeDtypeStruct((NP, D), jnp.float32),
        mesh=_MESH,
        scratch_types=[
            pltpu.VMEM((RPS, 128), jnp.int32),
            pltpu.VMEM((RPS, 128), jnp.int32),
            pltpu.VMEM((RPW,), jnp.int32),
            pltpu.VMEM((128, D), jnp.float32),
            pltpu.SemaphoreType.DMA,
            pltpu.VMEM_SHARED((NP2,), jnp.int32),
        ],
    )


def kernel(x, edge_index, W):
    src = edge_index[0].astype(jnp.int32)
    dst = edge_index[1].astype(jnp.int32)
    x = x.astype(jnp.float32)
    xp = jnp.pad(x, ((0, NP - N), (0, 0)))

    n_pad_a = NW * CHA * 128 - E
    pad_a = N + (jnp.arange(n_pad_a, dtype=jnp.int32) % (NP - N))
    dst_a = jnp.concatenate([dst, pad_a]).reshape(NW, CHA, 128)
    val_a = (jnp.arange(NW * CHA * 128) < E).astype(jnp.float32)
    val_a = val_a.reshape(NW, CHA, 128)

    loop = jnp.arange(N, dtype=jnp.int32)
    n_pad_c = NW * CHC * 128 - (E + N)
    pad_c = N + (jnp.arange(n_pad_c, dtype=jnp.int32) % (NP - N))
    src_c = jnp.concatenate([src, loop, pad_c]).reshape(NW, CHC, 128)
    dst_c = jnp.concatenate([dst, loop, pad_c]).reshape(NW, CHC, 128)

    zf = jnp.zeros((NP,), jnp.float32)
    z2 = jnp.zeros((NP, DH), jnp.float32)
    zi = jnp.zeros((NP2,), jnp.int32)
    u128 = jnp.triu(jnp.ones((128, 128), jnp.float32))
    s96 = jnp.tril(jnp.ones((NR, NR), jnp.float32), k=-1)
    vals = jnp.arange(NP, dtype=jnp.int32).reshape(NR, 128)

    hist = _make_hist()(dst_a, val_a, zf)
    feat_flat, normb_flat = _make_feat()(hist, xp.reshape(-1))
    feat = feat_flat.reshape(NP, D)
    part = _make_agg()(src_c, dst_c, feat[:, :DH], feat[:, DH:], z2)
    q0 = jnp.concatenate([part[0, 0], part[0, 1]], axis=1)
    q1 = jnp.concatenate([part[1, 0], part[1, 1]], axis=1)
    out_d = _matmul(q0, q1, normb_flat.reshape(NP, D), W.astype(jnp.float32))
    midx = _positions(hist.reshape(2, NR, 128), u128, s96)
    fin = _make_gather()(midx, vals, out_d, zi)
    return fin[:N]


# speedup vs baseline: 10.1390x; 1.2103x over previous
import jax
import jax.numpy as jnp
from jax import lax
from jax.experimental import pallas as pl
from jax.experimental.pallas import tpu as pltpu
from jax.experimental.pallas import tpu_sc as plsc

N = 10000
E = 320000
D = 128
NC, NS, L = 2, 16, 16
NW = NC * NS
NP = 12288
RPW = NP // NW
RPT = NP // NS

CHA = 79
CHC = 82

_MESH = plsc.VectorSubcoreMesh(
    core_axis_name="c", subcore_axis_name="s", num_cores=NC, num_subcores=NS)


def _wid():
    return lax.axis_index("s") * NC + lax.axis_index("c")


def _hist_body(dst_hbm, val_hbm, zf_hbm, hist_hbm, dst_v, val_v, bounce_v,
               hist_sh):
    c = lax.axis_index("c")
    s = lax.axis_index("s")
    w = _wid()

    @pl.when(s == 0)
    def _():
        pltpu.sync_copy(zf_hbm, hist_sh)

    pltpu.sync_copy(dst_hbm.at[w], dst_v)
    pltpu.sync_copy(val_hbm.at[w], val_v)
    plsc.subcore_barrier()

    def body(j, carry):
        pltpu.sync_copy(val_v.at[j], hist_sh.at[dst_v.at[j]], add=True)
        return carry

    lax.fori_loop(0, CHA, body, 0)
    plsc.subcore_barrier()

    @pl.when(s == 0)
    def _():
        pltpu.sync_copy(hist_sh, bounce_v)
        pltpu.sync_copy(bounce_v, hist_hbm.at[c])


def _make_hist():
    return pl.kernel(
        _hist_body,
        out_type=jax.ShapeDtypeStruct((NC, NP), jnp.float32),
        mesh=_MESH,
        scratch_types=[
            pltpu.VMEM((CHA, 128), jnp.int32),
            pltpu.VMEM((CHA, 128), jnp.float32),
            pltpu.VMEM((NP,), jnp.float32),
            pltpu.VMEM_SHARED((NP,), jnp.float32),
        ],
    )


def _feat_body(hist_hbm, x_hbm, feat_hbm, normb_hbm, h0_v, h1_v, norm_v, x_v,
               nb_v):
    w = _wid()
    base = w * RPW
    pltpu.sync_copy(hist_hbm.at[0].at[pl.ds(base, RPW)], h0_v)
    pltpu.sync_copy(hist_hbm.at[1].at[pl.ds(base, RPW)], h1_v)
    pltpu.sync_copy(x_hbm.at[pl.ds(base * D, RPW * D)], x_v)

    def nbody(v, carry):
        deg = h0_v[pl.ds(v * L, L)] + h1_v[pl.ds(v * L, L)] + 1.0
        i = lax.bitcast_convert_type(deg, jnp.int32)
        i = 0x5F3759DF - lax.shift_right_logical(i, 1)
        y = lax.bitcast_convert_type(i, jnp.float32)
        hh = deg * 0.5
        y = y * (1.5 - hh * y * y)
        y = y * (1.5 - hh * y * y)
        y = y * (1.5 - hh * y * y)
        norm_v[pl.ds(v * L, L)] = y
        return carry

    lax.fori_loop(0, RPW // L, nbody, 0)

    def gbody(g, carry):
        nv = norm_v[pl.ds(g * L, L)]
        for lane in range(L):
            idx = (jnp.zeros((L,), jnp.int32) + lane)[:, None]
            nb = lax.gather(
                nv, idx,
                lax.GatherDimensionNumbers(offset_dims=(),
                                           collapsed_slice_dims=(0,),
                                           start_index_map=(0,)),
                slice_sizes=(1,),
                mode=lax.GatherScatterMode.PROMISE_IN_BOUNDS)
            for k in range(D // L):
                off = (g * L + lane) * D + k * L
                x_v[pl.ds(off, L)] = x_v[pl.ds(off, L)] * nb
                nb_v[pl.ds(off, L)] = nb
        return carry

    lax.fori_loop(0, RPW // L, gbody, 0)

    pltpu.sync_copy(x_v, feat_hbm.at[pl.ds(base * D, RPW * D)])
    pltpu.sync_copy(nb_v, normb_hbm.at[pl.ds(base * D, RPW * D)])


def _make_feat():
    return pl.kernel(
        _feat_body,
        out_type=(jax.ShapeDtypeStruct((NP * D,), jnp.float32),
                  jax.ShapeDtypeStruct((NP * D,), jnp.float32)),
        mesh=_MESH,
        scratch_types=[
            pltpu.VMEM((RPW,), jnp.float32),
            pltpu.VMEM((RPW,), jnp.float32),
            pltpu.VMEM((RPW,), jnp.float32),
            pltpu.VMEM((RPW * D,), jnp.float32),
            pltpu.VMEM((RPW * D,), jnp.float32),
        ],
    )


DH = D // 2
NBUF = 4
CHT = 164


def _agg_body(src_hbm, dst_hbm, feat2_hbm, z_hbm, pout_hbm,
              src_v, dst_v, bufs, sems, agg_sh):
    c = lax.axis_index("c")
    s = lax.axis_index("s")

    @pl.when(s == 0)
    def _():
        pltpu.sync_copy(z_hbm, agg_sh)

    pltpu.sync_copy(src_hbm.at[s], src_v)
    pltpu.sync_copy(dst_hbm.at[s], dst_v)
    plsc.subcore_barrier()

    ftab = feat2_hbm.at[c]
    for b in range(NBUF):
        pltpu.async_copy(ftab.at[src_v.at[b]], bufs[b], sems[b])

    def body(t, carry):
        for b in range(NBUF):
            j = NBUF * t + b
            pltpu.make_async_copy(ftab.at[src_v.at[j]], bufs[b],
                                  sems[b]).wait()
            pltpu.sync_copy(bufs[b], agg_sh.at[dst_v.at[j]], add=True)

            @pl.when(j + NBUF < CHT)
            def _():
                pltpu.async_copy(ftab.at[src_v.at[j + NBUF]], bufs[b],
                                 sems[b])
        return carry

    lax.fori_loop(0, CHT // NBUF, body, 0)
    plsc.subcore_barrier()

    for q in range(RPT // 128):
        row0 = s * RPT + q * 128
        pltpu.sync_copy(agg_sh.at[pl.ds(row0, 128)], bufs[0])
        pltpu.sync_copy(bufs[0], pout_hbm.at[c].at[pl.ds(row0, 128)])


def _make_agg():
    return pl.kernel(
        _agg_body,
        out_type=jax.ShapeDtypeStruct((NC, NP, DH), jnp.float32),
        mesh=_MESH,
        scratch_types=[
            pltpu.VMEM((CHT, 128), jnp.int32),
            pltpu.VMEM((CHT, 128), jnp.int32),
            [pltpu.VMEM((128, DH), jnp.float32) for _ in range(NBUF)],
            [pltpu.SemaphoreType.DMA for _ in range(NBUF)],
            pltpu.VMEM_SHARED((NP, DH), jnp.float32),
        ],
        compiler_params=pltpu.CompilerParams(use_tc_tiling_on_sc=False),
    )


_BD = 512


def _mm_body(p_ref, nb_ref, w_ref, o_ref):
    a = p_ref[...] * nb_ref[...]
    o_ref[...] = lax.dot_general(a, w_ref[...], (((1,), (1,)), ((), ())),
                                 preferred_element_type=jnp.float32)


def _matmul(p, normb, w):
    return pl.pallas_call(
        _mm_body,
        grid=(NP // _BD,),
        in_specs=[
            pl.BlockSpec((_BD, D), lambda i: (i, 0)),
            pl.BlockSpec((_BD, D), lambda i: (i, 0)),
            pl.BlockSpec((D, D), lambda i: (0, 0)),
        ],
        out_specs=pl.BlockSpec((_BD, D), lambda i: (i, 0)),
        out_shape=jax.ShapeDtypeStruct((NP, D), jnp.float32),
    )(p, normb, w)


NR = NP // 128
NP2 = NP + 128


def _pos_body(h_ref, u_ref, s_ref, o_ref):
    p = ((h_ref[0] + h_ref[1]) > 0.0).astype(jnp.float32)
    incl = lax.dot_general(p, u_ref[...], (((1,), (0,)), ((), ())),
                           precision=lax.Precision.HIGHEST,
                           preferred_element_type=jnp.float32)
    rows = incl[:, 127:128]
    rowpref = lax.dot_general(s_ref[...], rows, (((1,), (0,)), ((), ())),
                              precision=lax.Precision.HIGHEST,
                              preferred_element_type=jnp.float32)
    excl = incl - p + rowpref
    col = lax.broadcasted_iota(jnp.int32, (NR, 128), 1)
    o_ref[...] = jnp.where(p > 0.0, excl.astype(jnp.int32), NP + col)


def _positions(h3, u128, s96):
    return pl.pallas_call(
        _pos_body,
        grid=(1,),
        in_specs=[
            pl.BlockSpec((2, NR, 128), lambda i: (0, 0, 0)),
            pl.BlockSpec((128, 128), lambda i: (0, 0)),
            pl.BlockSpec((NR, NR), lambda i: (0, 0)),
        ],
        out_specs=pl.BlockSpec((NR, 128), lambda i: (0, 0)),
        out_shape=jax.ShapeDtypeStruct((NR, 128), jnp.int32),
    )(h3, u128, s96)


RPS = 8


def _gather_body(midx_hbm, vals_hbm, src2d_hbm, zi_hbm, fin_hbm, pidx_v,
                 pval_v, idx_v, rows_v, sem, dn_sh):
    s = lax.axis_index("s")
    w = _wid()

    @pl.when(s == 0)
    def _():
        pltpu.sync_copy(zi_hbm, dn_sh)

    @pl.when(s < NR // RPS)
    def _():
        pltpu.sync_copy(midx_hbm.at[pl.ds(s * RPS, RPS)], pidx_v)
        pltpu.sync_copy(vals_hbm.at[pl.ds(s * RPS, RPS)], pval_v)

    plsc.subcore_barrier()

    @pl.when(s < NR // RPS)
    def _():
        for r in range(RPS):
            pltpu.sync_copy(pval_v.at[r], dn_sh.at[pidx_v.at[r]])

    plsc.subcore_barrier()

    base = w * RPW
    pltpu.sync_copy(dn_sh.at[pl.ds(base, RPW)], idx_v)
    for off in (0, 128, 256):
        pltpu.async_copy(src2d_hbm.at[idx_v.at[pl.ds(off, 128)]],
                         rows_v, sem).wait()
        pltpu.sync_copy(rows_v, fin_hbm.at[pl.ds(base + off, 128)])


def _make_gather():
    return pl.kernel(
        _gather_body,
        out_type=jax.ShapeDtypeStruct((NP, D), jnp.float32),
        mesh=_MESH,
        scratch_types=[
            pltpu.VMEM((RPS, 128), jnp.int32),
            pltpu.VMEM((RPS, 128), jnp.int32),
            pltpu.VMEM((RPW,), jnp.int32),
            pltpu.VMEM((128, D), jnp.float32),
            pltpu.SemaphoreType.DMA,
            pltpu.VMEM_SHARED((NP2,), jnp.int32),
        ],
    )


def kernel(x, edge_index, W):
    src = edge_index[0].astype(jnp.int32)
    dst = edge_index[1].astype(jnp.int32)
    x = x.astype(jnp.float32)
    xp = jnp.pad(x, ((0, NP - N), (0, 0)))

    n_pad_a = NW * CHA * 128 - E
    pad_a = N + (jnp.arange(n_pad_a, dtype=jnp.int32) % (NP - N))
    dst_a = jnp.concatenate([dst, pad_a]).reshape(NW, CHA, 128)
    val_a = (jnp.arange(NW * CHA * 128) < E).astype(jnp.float32)
    val_a = val_a.reshape(NW, CHA, 128)

    loop = jnp.arange(N, dtype=jnp.int32)
    n_pad_c = NS * CHT * 128 - (E + N)
    pad_c = N + (jnp.arange(n_pad_c, dtype=jnp.int32) % (NP - N))
    src_c = jnp.concatenate([src, loop, pad_c]).reshape(NS, CHT, 128)
    dst_c = jnp.concatenate([dst, loop, pad_c]).reshape(NS, CHT, 128)

    zf = jnp.zeros((NP,), jnp.float32)
    z2 = jnp.zeros((NP, DH), jnp.float32)
    zi = jnp.zeros((NP2,), jnp.int32)
    u128 = jnp.triu(jnp.ones((128, 128), jnp.float32))
    s96 = jnp.tril(jnp.ones((NR, NR), jnp.float32), k=-1)
    vals = jnp.arange(NP, dtype=jnp.int32).reshape(NR, 128)

    hist = _make_hist()(dst_a, val_a, zf)
    feat_flat, normb_flat = _make_feat()(hist, xp.reshape(-1))
    feat = feat_flat.reshape(NP, D)
    feat2 = jnp.stack([feat[:, :DH], feat[:, DH:]])
    part = _make_agg()(src_c, dst_c, feat2, z2)
    q = jnp.concatenate([part[0], part[1]], axis=1)
    out_d = _matmul(q, normb_flat.reshape(NP, D), W.astype(jnp.float32))
    midx = _positions(hist.reshape(2, NR, 128), u128, s96)
    fin = _make_gather()(midx, vals, out_d, zi)
    return fin[:N]


# speedup vs baseline: 13.9886x; 1.3797x over previous
import jax
import jax.numpy as jnp
from jax import lax
from jax.experimental import pallas as pl
from jax.experimental.pallas import tpu as pltpu
from jax.experimental.pallas import tpu_sc as plsc

N = 10000
E = 320000
D = 128
NC, NS, L = 2, 16, 16
NW = NC * NS
NP = 12288
RPW = NP // NW
RPT = NP // NS

CHA = 79
CHC = 82

_MESH = plsc.VectorSubcoreMesh(
    core_axis_name="c", subcore_axis_name="s", num_cores=NC, num_subcores=NS)


def _wid():
    return lax.axis_index("s") * NC + lax.axis_index("c")


def _hist_body(dst_hbm, val_hbm, zf_hbm, hist_hbm, dst_v, val_v, bounce_v,
               hist_sh):
    c = lax.axis_index("c")
    s = lax.axis_index("s")
    w = _wid()

    @pl.when(s == 0)
    def _():
        pltpu.sync_copy(zf_hbm, hist_sh)

    pltpu.sync_copy(dst_hbm.at[w], dst_v)
    pltpu.sync_copy(val_hbm.at[w], val_v)
    plsc.subcore_barrier()

    def body(j, carry):
        pltpu.sync_copy(val_v.at[j], hist_sh.at[dst_v.at[j]], add=True)
        return carry

    lax.fori_loop(0, CHA, body, 0)
    plsc.subcore_barrier()

    @pl.when(s == 0)
    def _():
        pltpu.sync_copy(hist_sh, bounce_v)
        pltpu.sync_copy(bounce_v, hist_hbm.at[c])


def _make_hist():
    return pl.kernel(
        _hist_body,
        out_type=jax.ShapeDtypeStruct((NC, NP), jnp.float32),
        mesh=_MESH,
        scratch_types=[
            pltpu.VMEM((CHA, 128), jnp.int32),
            pltpu.VMEM((CHA, 128), jnp.float32),
            pltpu.VMEM((NP,), jnp.float32),
            pltpu.VMEM_SHARED((NP,), jnp.float32),
        ],
    )


def _feat_body(hist_hbm, x_hbm, feat_hbm, normb_hbm, h0_v, h1_v, norm_v, x_v,
               nb_v):
    w = _wid()
    base = w * RPW
    pltpu.sync_copy(hist_hbm.at[0].at[pl.ds(base, RPW)], h0_v)
    pltpu.sync_copy(hist_hbm.at[1].at[pl.ds(base, RPW)], h1_v)
    pltpu.sync_copy(x_hbm.at[pl.ds(base * D, RPW * D)], x_v)

    def nbody(v, carry):
        deg = h0_v[pl.ds(v * L, L)] + h1_v[pl.ds(v * L, L)] + 1.0
        i = lax.bitcast_convert_type(deg, jnp.int32)
        i = 0x5F3759DF - lax.shift_right_logical(i, 1)
        y = lax.bitcast_convert_type(i, jnp.float32)
        hh = deg * 0.5
        y = y * (1.5 - hh * y * y)
        y = y * (1.5 - hh * y * y)
        y = y * (1.5 - hh * y * y)
        norm_v[pl.ds(v * L, L)] = y
        return carry

    lax.fori_loop(0, RPW // L, nbody, 0)

    def gbody(g, carry):
        nv = norm_v[pl.ds(g * L, L)]
        for lane in range(L):
            idx = (jnp.zeros((L,), jnp.int32) + lane)[:, None]
            nb = lax.gather(
                nv, idx,
                lax.GatherDimensionNumbers(offset_dims=(),
                                           collapsed_slice_dims=(0,),
                                           start_index_map=(0,)),
                slice_sizes=(1,),
                mode=lax.GatherScatterMode.PROMISE_IN_BOUNDS)
            for k in range(D // L):
                off = (g * L + lane) * D + k * L
                x_v[pl.ds(off, L)] = x_v[pl.ds(off, L)] * nb
                nb_v[pl.ds(off, L)] = nb
        return carry

    lax.fori_loop(0, RPW // L, gbody, 0)

    pltpu.sync_copy(x_v, feat_hbm.at[pl.ds(base * D, RPW * D)])
    pltpu.sync_copy(nb_v, normb_hbm.at[pl.ds(base * D, RPW * D)])


def _make_feat():
    return pl.kernel(
        _feat_body,
        out_type=(jax.ShapeDtypeStruct((NP * D,), jnp.float32),
                  jax.ShapeDtypeStruct((NP * D,), jnp.float32)),
        mesh=_MESH,
        scratch_types=[
            pltpu.VMEM((RPW,), jnp.float32),
            pltpu.VMEM((RPW,), jnp.float32),
            pltpu.VMEM((RPW,), jnp.float32),
            pltpu.VMEM((RPW * D,), jnp.float32),
            pltpu.VMEM((RPW * D,), jnp.float32),
        ],
    )


DH = D // 2
NBUF = 4
CHT = 164


def _agg_body(src_hbm, dst_hbm, feat2_hbm, z_hbm, pout_hbm,
              src_v, dst_v, bufs, sems, agg_sh):
    c = lax.axis_index("c")
    s = lax.axis_index("s")

    @pl.when(s == 0)
    def _():
        pltpu.sync_copy(z_hbm, agg_sh)

    pltpu.sync_copy(src_hbm.at[s], src_v)
    pltpu.sync_copy(dst_hbm.at[s], dst_v)
    plsc.subcore_barrier()

    ftab = feat2_hbm.at[c]
    for b in range(NBUF):
        pltpu.async_copy(ftab.at[src_v.at[b]], bufs[b], sems[b])

    def body(t, carry):
        for b in range(NBUF):
            j = NBUF * t + b
            pltpu.make_async_copy(ftab.at[src_v.at[j]], bufs[b],
                                  sems[b]).wait()
            pltpu.sync_copy(bufs[b], agg_sh.at[dst_v.at[j]], add=True)

            @pl.when(j + NBUF < CHT)
            def _():
                pltpu.async_copy(ftab.at[src_v.at[j + NBUF]], bufs[b],
                                 sems[b])
        return carry

    lax.fori_loop(0, CHT // NBUF, body, 0)
    plsc.subcore_barrier()

    for q in range(RPT // 128):
        row0 = s * RPT + q * 128
        pltpu.sync_copy(agg_sh.at[pl.ds(row0, 128)], bufs[0])
        pltpu.sync_copy(bufs[0], pout_hbm.at[c].at[pl.ds(row0, 128)])


def _make_agg():
    return pl.kernel(
        _agg_body,
        out_type=jax.ShapeDtypeStruct((NC, NP, DH), jnp.float32),
        mesh=_MESH,
        scratch_types=[
            pltpu.VMEM((CHT, 128), jnp.int32),
            pltpu.VMEM((CHT, 128), jnp.int32),
            [pltpu.VMEM((128, DH), jnp.float32) for _ in range(NBUF)],
            [pltpu.SemaphoreType.DMA for _ in range(NBUF)],
            pltpu.VMEM_SHARED((NP, DH), jnp.float32),
        ],
        compiler_params=pltpu.CompilerParams(use_tc_tiling_on_sc=False),
    )


_BD = 512


def _mm_body(p_ref, nb_ref, w_ref, o_ref):
    a = p_ref[...] * nb_ref[...]
    o_ref[...] = lax.dot_general(a, w_ref[...], (((1,), (1,)), ((), ())),
                                 preferred_element_type=jnp.float32)


def _matmul(p, normb, w):
    return pl.pallas_call(
        _mm_body,
        grid=(NP // _BD,),
        in_specs=[
            pl.BlockSpec((_BD, D), lambda i: (i, 0)),
            pl.BlockSpec((_BD, D), lambda i: (i, 0)),
            pl.BlockSpec((D, D), lambda i: (0, 0)),
        ],
        out_specs=pl.BlockSpec((_BD, D), lambda i: (i, 0)),
        out_shape=jax.ShapeDtypeStruct((NP, D), jnp.float32),
    )(p, normb, w)


NR = NP // 128
NP2 = NP + 128


def _pos_body(h_ref, u_ref, s_ref, o_ref):
    p = ((h_ref[0] + h_ref[1]) > 0.0).astype(jnp.float32)
    incl = lax.dot_general(p, u_ref[...], (((1,), (0,)), ((), ())),
                           precision=lax.Precision.HIGHEST,
                           preferred_element_type=jnp.float32)
    rows = incl[:, 127:128]
    rowpref = lax.dot_general(s_ref[...], rows, (((1,), (0,)), ((), ())),
                              precision=lax.Precision.HIGHEST,
                              preferred_element_type=jnp.float32)
    excl = incl - p + rowpref
    col = lax.broadcasted_iota(jnp.int32, (NR, 128), 1)
    o_ref[...] = jnp.where(p > 0.0, excl.astype(jnp.int32), NP + col)


def _positions(h3, u128, s96):
    return pl.pallas_call(
        _pos_body,
        grid=(1,),
        in_specs=[
            pl.BlockSpec((2, NR, 128), lambda i: (0, 0, 0)),
            pl.BlockSpec((128, 128), lambda i: (0, 0)),
            pl.BlockSpec((NR, NR), lambda i: (0, 0)),
        ],
        out_specs=pl.BlockSpec((NR, 128), lambda i: (0, 0)),
        out_shape=jax.ShapeDtypeStruct((NR, 128), jnp.int32),
    )(h3, u128, s96)


RPS = 8


def _gather_body(midx_hbm, vals_hbm, src2d_hbm, zi_hbm, fin_hbm, pidx_v,
                 pval_v, idx_v, rows_v, sem, dn_sh):
    s = lax.axis_index("s")
    w = _wid()

    @pl.when(s == 0)
    def _():
        pltpu.sync_copy(zi_hbm, dn_sh)

    @pl.when(s < NR // RPS)
    def _():
        pltpu.sync_copy(midx_hbm.at[pl.ds(s * RPS, RPS)], pidx_v)
        pltpu.sync_copy(vals_hbm.at[pl.ds(s * RPS, RPS)], pval_v)

    plsc.subcore_barrier()

    @pl.when(s < NR // RPS)
    def _():
        for r in range(RPS):
            pltpu.sync_copy(pval_v.at[r], dn_sh.at[pidx_v.at[r]])

    plsc.subcore_barrier()

    base = w * RPW
    pltpu.sync_copy(dn_sh.at[pl.ds(base, RPW)], idx_v)
    for off in (0, 128, 256):
        pltpu.async_copy(src2d_hbm.at[idx_v.at[pl.ds(off, 128)]],
                         rows_v, sem).wait()
        pltpu.sync_copy(rows_v, fin_hbm.at[pl.ds(base + off, 128)])


def _make_gather():
    return pl.kernel(
        _gather_body,
        out_type=jax.ShapeDtypeStruct((NP, D), jnp.float32),
        mesh=_MESH,
        scratch_types=[
            pltpu.VMEM((RPS, 128), jnp.int32),
            pltpu.VMEM((RPS, 128), jnp.int32),
            pltpu.VMEM((RPW,), jnp.int32),
            pltpu.VMEM((128, D), jnp.float32),
            pltpu.SemaphoreType.DMA,
            pltpu.VMEM_SHARED((NP2,), jnp.int32),
        ],
    )


def kernel(x, edge_index, W):
    src = edge_index[0].astype(jnp.int32)
    dst = edge_index[1].astype(jnp.int32)
    x = x.astype(jnp.float32)
    xp = jnp.pad(x, ((0, NP - N), (0, 0)))

    n_pad_a = NW * CHA * 128 - E
    pad_a = N + (jnp.arange(n_pad_a, dtype=jnp.int32) % (NP - N))
    dst_a = jnp.concatenate([dst, pad_a]).reshape(NW, CHA, 128)
    val_a = (jnp.arange(NW * CHA * 128) < E).astype(jnp.float32)
    val_a = val_a.reshape(NW, CHA, 128)

    loop = jnp.arange(N, dtype=jnp.int32)
    n_pad_c = NS * CHT * 128 - (E + N)
    pad_c = N + (jnp.arange(n_pad_c, dtype=jnp.int32) % (NP - N))
    src_c = jnp.concatenate([src, loop, pad_c]).reshape(NS, CHT, 128)
    dst_c = jnp.concatenate([dst, loop, pad_c]).reshape(NS, CHT, 128)

    zf = jnp.zeros((NP,), jnp.float32)
    z2 = jnp.zeros((NP, DH), jnp.float32)
    zi = jnp.concatenate([
        jnp.zeros((N,), jnp.int32),
        jnp.arange(NP2 - N, dtype=jnp.int32) % N,
    ])
    u128 = jnp.triu(jnp.ones((128, 128), jnp.float32))
    s96 = jnp.tril(jnp.ones((NR, NR), jnp.float32), k=-1)
    vals = jnp.arange(NP, dtype=jnp.int32).reshape(NR, 128)

    hist = _make_hist()(dst_a, val_a, zf)
    feat_flat, normb_flat = _make_feat()(hist, xp.reshape(-1))
    feat = feat_flat.reshape(NP, D)
    feat2 = jnp.stack([feat[:, :DH], feat[:, DH:]])
    part = _make_agg()(src_c, dst_c, feat2, z2)
    q = jnp.concatenate([part[0], part[1]], axis=1)
    out_d = _matmul(q, normb_flat.reshape(NP, D), W.astype(jnp.float32))
    midx = _positions(hist.reshape(2, NR, 128), u128, s96)
    fin = _make_gather()(midx, vals, out_d, zi)
    return fin[:N]
